# Initial kernel scaffold; baseline (speedup 1.0000x reference)
import jax
import jax.numpy as jnp
from jax.experimental import pallas as pl


def kernel(intent_ids, table):
    raise NotImplementedError("write your pallas kernel here")


# speedup vs baseline: 4.7518x; 4.7518x over previous
import functools
import jax
import jax.numpy as jnp
from jax import lax
from jax.experimental import pallas as pl
from jax.experimental.pallas import tpu as pltpu
from jax.experimental.pallas import tpu_sc as plsc

NC = 2
NS = 16
NW = NC * NS
CHUNK = 128
SUB = 4
SUPER = CHUNK * SUB


def _gather_kernel(B, V, D, b_per_w, n_super):
    mesh = plsc.VectorSubcoreMesh(core_axis_name="c", subcore_axis_name="s")

    @functools.partial(
        pl.kernel,
        out_type=jax.ShapeDtypeStruct((B, D), jnp.float32),
        mesh=mesh,
        scratch_types=[
            pltpu.VMEM((SUB, CHUNK), jnp.int32),
            pltpu.VMEM((SUPER, D), jnp.float32),
            pltpu.SemaphoreType.DMA,
        ],
        compiler_params=pltpu.CompilerParams(use_tc_tiling_on_sc=False),
    )
    def k(idx_hbm, table_hbm, out_hbm, idx_v, rows_v, gsem):
        wid = lax.axis_index("s") * NC + lax.axis_index("c")
        row0 = wid * (b_per_w // CHUNK)
        out0 = wid * b_per_w

        def body(g, carry):
            pltpu.sync_copy(idx_hbm.at[pl.ds(row0 + g * SUB, SUB)], idx_v)
            descs = []
            for j in range(SUB):
                descs.append(
                    pltpu.async_copy(
                        table_hbm.at[idx_v.at[j]],
                        rows_v.at[pl.ds(j * CHUNK, CHUNK)],
                        gsem,
                    )
                )
            for d in descs:
                d.wait()
            pltpu.sync_copy(rows_v, out_hbm.at[pl.ds(out0 + g * SUPER, SUPER)])
            return carry

        lax.fori_loop(0, n_super, body, 0)

    return k


def kernel(intent_ids, table):
    B_, S = intent_ids.shape
    V, D = table.shape
    B = B_ * S
    assert B % (NW * SUPER) == 0
    b_per_w = B // NW
    n_super = b_per_w // SUPER

    idx2d = intent_ids.reshape(B // CHUNK, CHUNK)
    out = _gather_kernel(B, V, D, b_per_w, n_super)(idx2d, table)
    return out.reshape(B_, S, D)


# speedup vs baseline: 5.1581x; 1.0855x over previous
import functools
import jax
import jax.numpy as jnp
from jax import lax
from jax.experimental import pallas as pl
from jax.experimental.pallas import tpu as pltpu
from jax.experimental.pallas import tpu_sc as plsc

NC = 2
NS = 16
NW = NC * NS
CHUNK = 128
SUB = 4
SUPER = CHUNK * SUB


def _gather_kernel(B, V, D, b_per_w, n_super):
    mesh = plsc.VectorSubcoreMesh(core_axis_name="c", subcore_axis_name="s")

    @functools.partial(
        pl.kernel,
        out_type=jax.ShapeDtypeStruct((B, D), jnp.float32),
        mesh=mesh,
        scratch_types=[
            pltpu.VMEM((2, SUB, CHUNK), jnp.int32),
            pltpu.VMEM((2, SUPER, D), jnp.float32),
            pltpu.SemaphoreType.DMA,
            pltpu.SemaphoreType.DMA,
            pltpu.SemaphoreType.DMA,
            pltpu.SemaphoreType.DMA,
            pltpu.SemaphoreType.DMA,
            pltpu.SemaphoreType.DMA,
        ],
        compiler_params=pltpu.CompilerParams(use_tc_tiling_on_sc=False),
    )
    def k(idx_hbm, table_hbm, out_hbm, idx_v, rows_v, g0, g1, s0, s1, i0, i1):
        gsem = (g0, g1)
        ssem = (s0, s1)
        isem = (i0, i1)
        wid = lax.axis_index("s") * NC + lax.axis_index("c")
        irow0 = wid * (b_per_w // CHUNK)
        out0 = wid * b_per_w

        def idx_copy(g, slot):
            return pltpu.make_async_copy(
                idx_hbm.at[pl.ds(irow0 + g * SUB, SUB)],
                idx_v.at[slot],
                isem[slot],
            )

        def gather_copies(slot):
            return [
                pltpu.make_async_copy(
                    table_hbm.at[idx_v.at[slot, j]],
                    rows_v.at[slot, pl.ds(j * CHUNK, CHUNK)],
                    gsem[slot],
                )
                for j in range(SUB)
            ]

        def store_copy(g, slot):
            return pltpu.make_async_copy(
                rows_v.at[slot],
                out_hbm.at[pl.ds(out0 + g * SUPER, SUPER)],
                ssem[slot],
            )

        def stage(g, b, first=False, prefetch=True):
            nb = 1 - b
            for d in gather_copies(b):
                d.wait()
            store_copy(g, b).start()
            idx_copy(g + 1, nb).wait()
            if not first:
                store_copy(g - 1, nb).wait()
            for d in gather_copies(nb):
                d.start()
            if prefetch:
                idx_copy(g + 2, b).start()

        idx_copy(0, 0).start()
        idx_copy(0, 0).wait()
        for d in gather_copies(0):
            d.start()
        idx_copy(1, 1).start()

        stage(0, 0, first=True)
        stage(1, 1)

        def body(h, carry):
            for b in range(2):
                stage(2 * h + b, b)
            return carry

        lax.fori_loop(1, n_super // 2 - 1, body, 0)

        stage(n_super - 2, 0, prefetch=False)
        for d in gather_copies(1):
            d.wait()
        store_copy(n_super - 1, 1).start()

        store_copy(n_super - 2, 0).wait()
        store_copy(n_super - 1, 1).wait()

    return k


def kernel(intent_ids, table):
    B_, S = intent_ids.shape
    V, D = table.shape
    B = B_ * S
    assert B % (NW * SUPER) == 0
    b_per_w = B // NW
    n_super = b_per_w // SUPER
    assert n_super >= 4 and n_super % 2 == 0

    idx2d = intent_ids.reshape(B // CHUNK, CHUNK)
    out = _gather_kernel(B, V, D, b_per_w, n_super)(idx2d, table)
    return out.reshape(B_, S, D)


# speedup vs baseline: 5.1716x; 1.0026x over previous
import functools
import jax
import jax.numpy as jnp
from jax import lax
from jax.experimental import pallas as pl
from jax.experimental.pallas import tpu as pltpu
from jax.experimental.pallas import tpu_sc as plsc

NC = 2
NS = 16
NW = NC * NS
BSTAGE = 4
SPLITS = (0, 104, 200)


def _gather_kernel(B_, S, V, D, bat_per_w, n_stage):
    mesh = plsc.VectorSubcoreMesh(core_axis_name="c", subcore_axis_name="s")
    rows_stage = BSTAGE * S

    @functools.partial(
        pl.kernel,
        out_type=jax.ShapeDtypeStruct((B_, S, D), jnp.float32),
        mesh=mesh,
        scratch_types=[
            pltpu.VMEM((2, BSTAGE, S), jnp.int32),
            pltpu.VMEM((2, BSTAGE, S, D), jnp.float32),
            pltpu.SemaphoreType.DMA,
            pltpu.SemaphoreType.DMA,
            pltpu.SemaphoreType.DMA,
            pltpu.SemaphoreType.DMA,
            pltpu.SemaphoreType.DMA,
            pltpu.SemaphoreType.DMA,
        ],
        compiler_params=pltpu.CompilerParams(use_tc_tiling_on_sc=False),
    )
    def k(idx_hbm, table_hbm, out_hbm, idx_v, rows_v, g0, g1, s0, s1, i0, i1):
        gsem = (g0, g1)
        ssem = (s0, s1)
        isem = (i0, i1)
        wid = lax.axis_index("s") * NC + lax.axis_index("c")
        bat0 = wid * bat_per_w

        def idx_copy(g, slot):
            return pltpu.make_async_copy(
                idx_hbm.at[pl.ds(bat0 + g * BSTAGE, BSTAGE)],
                idx_v.at[slot],
                isem[slot],
            )

        def gather_copies(slot):
            out = []
            for i in range(BSTAGE):
                for c in range(len(SPLITS) - 1):
                    lo, hi = SPLITS[c], SPLITS[c + 1]
                    out.append(
                        pltpu.make_async_copy(
                            table_hbm.at[idx_v.at[slot, i, pl.ds(lo, hi - lo)]],
                            rows_v.at[slot, i, pl.ds(lo, hi - lo)],
                            gsem[slot],
                        )
                    )
            return out

        def store_copy(g, slot):
            return pltpu.make_async_copy(
                rows_v.at[slot],
                out_hbm.at[pl.ds(bat0 + g * BSTAGE, BSTAGE)],
                ssem[slot],
            )

        def stage(g, b, first=False, prefetch=True):
            nb = 1 - b
            for d in gather_copies(b):
                d.wait()
            store_copy(g, b).start()
            idx_copy(g + 1, nb).wait()
            if not first:
                store_copy(g - 1, nb).wait()
            for d in gather_copies(nb):
                d.start()
            if prefetch:
                idx_copy(g + 2, b).start()

        idx_copy(0, 0).start()
        idx_copy(0, 0).wait()
        for d in gather_copies(0):
            d.start()
        idx_copy(1, 1).start()

        stage(0, 0, first=True)
        stage(1, 1)

        def body(h, carry):
            for b in range(2):
                stage(2 * h + b, b)
            return carry

        lax.fori_loop(1, n_stage // 2 - 1, body, 0)

        stage(n_stage - 2, 0, prefetch=False)
        for d in gather_copies(1):
            d.wait()
        store_copy(n_stage - 1, 1).start()

        store_copy(n_stage - 2, 0).wait()
        store_copy(n_stage - 1, 1).wait()

    return k


def kernel(intent_ids, table):
    B_, S = intent_ids.shape
    V, D = table.shape
    assert B_ % (NW * BSTAGE) == 0 and S == SPLITS[-1]
    bat_per_w = B_ // NW
    n_stage = bat_per_w // BSTAGE
    assert n_stage >= 4 and n_stage % 2 == 0

    return _gather_kernel(B_, S, V, D, bat_per_w, n_stage)(intent_ids, table)
